# baseline (speedup 1.0000x reference)
import jax
import jax.numpy as jnp
from jax.experimental import pallas as pl

IN_N = 2048
BLOCK_M = 512


def _copy_body(x_ref, o_ref):
    o_ref[...] = x_ref[...]


def kernel(input):
    x = input.reshape(-1, IN_N)
    m, n = x.shape
    return pl.pallas_call(
        _copy_body,
        grid=(m // BLOCK_M,),
        in_specs=[pl.BlockSpec((BLOCK_M, n), lambda i: (i, 0))],
        out_specs=pl.BlockSpec((BLOCK_M, n), lambda i: (i, 0)),
        out_shape=jax.ShapeDtypeStruct((m, n), x.dtype),
    )(x)
